# baseline (speedup 1.0000x reference)
import functools

import jax
import jax.numpy as jnp
from jax import lax
from jax.experimental import pallas as pl
from jax.experimental.pallas import tpu as pltpu
from jax.experimental.pallas import tpu_sc as plsc

N_NODES = 10000
N_EDGES = 320000
D_FEAT = 128
D_EDGE = 16
D_OUT = 16

NC, NS = 2, 16
NW = NC * NS
E_PER_W = N_EDGES // NW
CHUNK = 2000
N_CHUNKS = E_PER_W // CHUNK
SUB = 400
N_SUBS = CHUNK // SUB

FIN_BLK = 8192


def _proj_body(x_ref, w_ref, xs_ref, xr_ref):
    p = jnp.dot(x_ref[...], w_ref[...], preferred_element_type=jnp.float32)
    xs_ref[...] = p[:, :D_OUT]
    xr_ref[...] = p[:, D_OUT:]


def _node_proj(x, w_sr):
    return pl.pallas_call(
        _proj_body,
        out_shape=(
            jax.ShapeDtypeStruct((N_NODES, D_OUT), jnp.float32),
            jax.ShapeDtypeStruct((N_NODES, D_OUT), jnp.float32),
        ),
    )(x, w_sr)


def _final_body(eat_ref, wet_ref, b_ref, *refs):
    o_ref = refs[-1]
    base = (
        jnp.dot(wet_ref[...], eat_ref[...], preferred_element_type=jnp.float32)
        + b_ref[...]
    )
    for j in range(D_OUT):
        o_ref[j:j + 1, :] = base[j:j + 1, :] + refs[j][...][None, :]


def _final(eat, wet, bcol, tmps):
    return pl.pallas_call(
        _final_body,
        grid=(pl.cdiv(N_EDGES, FIN_BLK),),
        in_specs=[
            pl.BlockSpec((D_OUT, FIN_BLK), lambda i: (0, i)),
            pl.BlockSpec((D_EDGE, D_OUT), lambda i: (0, 0)),
            pl.BlockSpec((D_OUT, 1), lambda i: (0, 0)),
        ] + [pl.BlockSpec((FIN_BLK,), lambda i: (i,)) for _ in range(D_OUT)],
        out_specs=pl.BlockSpec((D_OUT, FIN_BLK), lambda i: (0, i)),
        out_shape=jax.ShapeDtypeStruct((D_OUT, N_EDGES), jnp.float32),
    )(eat, wet, bcol, *tmps)


_SC_MESH = plsc.VectorSubcoreMesh(core_axis_name="c", subcore_axis_name="s")


@functools.partial(
    pl.kernel,
    out_type=tuple(jax.ShapeDtypeStruct((N_EDGES,), jnp.float32)
                   for _ in range(D_OUT)),
    mesh=_SC_MESH,
    compiler_params=pltpu.CompilerParams(
        use_tc_tiling_on_sc=False, needs_layout_passes=False),
    scratch_types=[
        pltpu.VMEM((2, CHUNK), jnp.int32),
        pltpu.VMEM((2, CHUNK), jnp.int32),
        pltpu.VMEM((2, SUB, D_OUT), jnp.float32),
        pltpu.VMEM((2, SUB, D_OUT), jnp.float32),
        pltpu.VMEM((D_OUT, CHUNK), jnp.float32),
        pltpu.SemaphoreType.DMA,
        pltpu.SemaphoreType.DMA,
        pltpu.SemaphoreType.DMA,
    ],
)
def _sc_edge(xs_hbm, xr_hbm, eidx_hbm, *refs):
    out_hbms = refs[:D_OUT]
    (idx_s, idx_r, bufs, bufr, buft, sems, semr, semo) = refs[D_OUT:]
    wid = lax.axis_index("s") * NC + lax.axis_index("c")
    row0 = wid * E_PER_W
    lane = lax.iota(jnp.int32, 16)
    diags = [(lane + j) % 16 for j in range(D_OUT)]

    def issue_gathers(ci, si):
        ip = ci % 2
        sp = si % 2
        sl = pl.ds(si * SUB, SUB)
        pltpu.async_copy(xs_hbm.at[idx_s.at[ip, sl]], bufs.at[sp], sems)
        pltpu.async_copy(xr_hbm.at[idx_r.at[ip, sl]], bufr.at[sp], semr)

    def wait_gathers(si):
        sp = si % 2
        pltpu.make_async_copy(xs_hbm.at[pl.ds(0, SUB)], bufs.at[sp], sems).wait()
        pltpu.make_async_copy(xr_hbm.at[pl.ds(0, SUB)], bufr.at[sp], semr).wait()

    pltpu.sync_copy(eidx_hbm.at[0, pl.ds(row0, CHUNK)], idx_s.at[0])
    pltpu.sync_copy(eidx_hbm.at[1, pl.ds(row0, CHUNK)], idx_r.at[0])
    issue_gathers(0, 0)

    for ci in range(N_CHUNKS):
        p = ci % 2
        off = row0 + ci * CHUNK
        if ci + 1 < N_CHUNKS:
            off1 = off + CHUNK
            pltpu.sync_copy(eidx_hbm.at[0, pl.ds(off1, CHUNK)],
                            idx_s.at[1 - p])
            pltpu.sync_copy(eidx_hbm.at[1, pl.ds(off1, CHUNK)],
                            idx_r.at[1 - p])
        if ci > 0:
            for j in range(D_OUT):
                pltpu.make_async_copy(buft.at[j],
                                      out_hbms[j].at[pl.ds(0, CHUNK)],
                                      semo).wait()
        for si in range(N_SUBS):
            sp = si % 2
            wait_gathers(si)
            if si + 1 < N_SUBS:
                issue_gathers(ci, si + 1)
            elif ci + 1 < N_CHUNKS:
                issue_gathers(ci + 1, 0)

            def t_body(g, c, _sp=sp, _si=si):
                rows = g * 16 + lane
                vs = [plsc.load_gather(bufs.at[_sp], [rows, d])
                      + plsc.load_gather(bufr.at[_sp], [rows, d])
                      for d in diags]
                cols = _si * SUB + rows
                for j in range(D_OUT):
                    plsc.store_scatter(buft, [diags[j], cols], vs[j])
                return c

            lax.fori_loop(0, SUB // 16, t_body, 0)
        for j in range(D_OUT):
            pltpu.async_copy(buft.at[j], out_hbms[j].at[pl.ds(off, CHUNK)],
                             semo)
    for j in range(D_OUT):
        pltpu.make_async_copy(buft.at[j], out_hbms[j].at[pl.ds(0, CHUNK)],
                              semo).wait()


def kernel(x, edge_index, edge_attr, W, b):
    w_sr = jnp.concatenate([W[:D_FEAT], W[D_FEAT:2 * D_FEAT]], axis=1)
    wet = W[2 * D_FEAT:].T
    bcol = b.reshape(D_OUT, 1)

    xs, xr = _node_proj(x, w_sr)
    tmps = _sc_edge(xs, xr, edge_index)
    out_t = _final(edge_attr.T, wet, bcol, tmps)
    return out_t.T


# speedup vs baseline: 1.1431x; 1.1431x over previous
import functools

import jax
import jax.numpy as jnp
from jax import lax
from jax.experimental import pallas as pl
from jax.experimental.pallas import tpu as pltpu
from jax.experimental.pallas import tpu_sc as plsc

N_NODES = 10000
N_EDGES = 320000
D_FEAT = 128
D_EDGE = 16
D_OUT = 16

NC, NS = 2, 16
NW = NC * NS
E_PER_W = N_EDGES // NW
CHUNK = 2000
N_CHUNKS = E_PER_W // CHUNK
SUB = 400
N_SUBS = CHUNK // SUB

FIN_BLK = 32768


def _proj_body(x_ref, w_ref, xs_ref, xr_ref):
    p = jnp.dot(x_ref[...], w_ref[...], preferred_element_type=jnp.float32)
    xs_ref[...] = p[:, :D_OUT]
    xr_ref[...] = p[:, D_OUT:]


def _node_proj(x, w_sr):
    return pl.pallas_call(
        _proj_body,
        out_shape=(
            jax.ShapeDtypeStruct((N_NODES, D_OUT), jnp.float32),
            jax.ShapeDtypeStruct((N_NODES, D_OUT), jnp.float32),
        ),
    )(x, w_sr)


def _final_body(eat_ref, wet_ref, b_ref, *refs):
    o_ref = refs[-1]
    base = (
        jnp.dot(wet_ref[...], eat_ref[...], preferred_element_type=jnp.float32)
        + b_ref[...]
    )
    for j in range(D_OUT):
        o_ref[j:j + 1, :] = base[j:j + 1, :] + refs[j][...][None, :]


def _final(eat, wet, bcol, tmps):
    return pl.pallas_call(
        _final_body,
        grid=(pl.cdiv(N_EDGES, FIN_BLK),),
        in_specs=[
            pl.BlockSpec((D_OUT, FIN_BLK), lambda i: (0, i)),
            pl.BlockSpec((D_EDGE, D_OUT), lambda i: (0, 0)),
            pl.BlockSpec((D_OUT, 1), lambda i: (0, 0)),
        ] + [pl.BlockSpec((FIN_BLK,), lambda i: (i,)) for _ in range(D_OUT)],
        out_specs=pl.BlockSpec((D_OUT, FIN_BLK), lambda i: (0, i)),
        out_shape=jax.ShapeDtypeStruct((D_OUT, N_EDGES), jnp.float32),
    )(eat, wet, bcol, *tmps)


_SC_MESH = plsc.VectorSubcoreMesh(core_axis_name="c", subcore_axis_name="s")


@functools.partial(
    pl.kernel,
    out_type=tuple(jax.ShapeDtypeStruct((N_EDGES,), jnp.float32)
                   for _ in range(D_OUT)),
    mesh=_SC_MESH,
    compiler_params=pltpu.CompilerParams(
        use_tc_tiling_on_sc=False, needs_layout_passes=False),
    scratch_types=[
        pltpu.VMEM((2, CHUNK), jnp.int32),
        pltpu.VMEM((2, CHUNK), jnp.int32),
        pltpu.VMEM((2, SUB, D_OUT), jnp.float32),
        pltpu.VMEM((2, SUB, D_OUT), jnp.float32),
        pltpu.VMEM((D_OUT, CHUNK), jnp.float32),
        pltpu.SemaphoreType.DMA,
        pltpu.SemaphoreType.DMA,
        pltpu.SemaphoreType.DMA,
    ],
)
def _sc_edge(xs_hbm, xr_hbm, eidx_hbm, *refs):
    out_hbms = refs[:D_OUT]
    (idx_s, idx_r, bufs, bufr, buft, sems, semr, semo) = refs[D_OUT:]
    wid = lax.axis_index("s") * NC + lax.axis_index("c")
    row0 = wid * E_PER_W
    lane = lax.iota(jnp.int32, 16)
    diags = [(lane + j) % 16 for j in range(D_OUT)]

    def issue_gathers(ci, si):
        ip = ci % 2
        sp = si % 2
        sl = pl.ds(si * SUB, SUB)
        pltpu.async_copy(xs_hbm.at[idx_s.at[ip, sl]], bufs.at[sp], sems)
        pltpu.async_copy(xr_hbm.at[idx_r.at[ip, sl]], bufr.at[sp], semr)

    def wait_gathers(si):
        sp = si % 2
        pltpu.make_async_copy(xs_hbm.at[pl.ds(0, SUB)], bufs.at[sp], sems).wait()
        pltpu.make_async_copy(xr_hbm.at[pl.ds(0, SUB)], bufr.at[sp], semr).wait()

    pltpu.sync_copy(eidx_hbm.at[0, pl.ds(row0, CHUNK)], idx_s.at[0])
    pltpu.sync_copy(eidx_hbm.at[1, pl.ds(row0, CHUNK)], idx_r.at[0])
    issue_gathers(0, 0)

    for ci in range(N_CHUNKS):
        p = ci % 2
        off = row0 + ci * CHUNK
        if ci + 1 < N_CHUNKS:
            off1 = off + CHUNK
            pltpu.sync_copy(eidx_hbm.at[0, pl.ds(off1, CHUNK)],
                            idx_s.at[1 - p])
            pltpu.sync_copy(eidx_hbm.at[1, pl.ds(off1, CHUNK)],
                            idx_r.at[1 - p])
        if ci > 0:
            for j in range(D_OUT):
                pltpu.make_async_copy(buft.at[j],
                                      out_hbms[j].at[pl.ds(0, CHUNK)],
                                      semo).wait()
        for si in range(N_SUBS):
            sp = si % 2
            wait_gathers(si)
            if si + 1 < N_SUBS:
                issue_gathers(ci, si + 1)
            elif ci + 1 < N_CHUNKS:
                issue_gathers(ci + 1, 0)

            def t_body(g, c, _sp=sp, _si=si):
                rows = g * 16 + lane
                vs = [plsc.load_gather(bufs.at[_sp], [rows, d])
                      + plsc.load_gather(bufr.at[_sp], [rows, d])
                      for d in diags]
                cols = _si * SUB + rows
                for j in range(D_OUT):
                    plsc.store_scatter(buft, [diags[j], cols], vs[j])
                return c

            lax.fori_loop(0, SUB // 16, t_body, 0)
        for j in range(D_OUT):
            pltpu.async_copy(buft.at[j], out_hbms[j].at[pl.ds(off, CHUNK)],
                             semo)
    for j in range(D_OUT):
        pltpu.make_async_copy(buft.at[j], out_hbms[j].at[pl.ds(0, CHUNK)],
                              semo).wait()


def kernel(x, edge_index, edge_attr, W, b):
    w_sr = jnp.concatenate([W[:D_FEAT], W[D_FEAT:2 * D_FEAT]], axis=1)
    wet = W[2 * D_FEAT:].T
    bcol = b.reshape(D_OUT, 1)

    xs, xr = _node_proj(x, w_sr)
    tmps = _sc_edge(xs, xr, edge_index)
    out_t = _final(edge_attr.T, wet, bcol, tmps)
    return out_t.T


# speedup vs baseline: 1.1587x; 1.0136x over previous
import functools

import jax
import jax.numpy as jnp
from jax import lax
from jax.experimental import pallas as pl
from jax.experimental.pallas import tpu as pltpu
from jax.experimental.pallas import tpu_sc as plsc

N_NODES = 10000
N_EDGES = 320000
D_FEAT = 128
D_EDGE = 16
D_OUT = 16

NC, NS = 2, 16
NW = NC * NS
E_PER_W = N_EDGES // NW
CHUNK = 2000
N_CHUNKS = E_PER_W // CHUNK
SUB = 400
N_SUBS = CHUNK // SUB

FIN_BLK = 65536


def _proj_body(x_ref, w_ref, xs_ref, xr_ref):
    p = jnp.dot(x_ref[...], w_ref[...], preferred_element_type=jnp.float32)
    xs_ref[...] = p[:, :D_OUT]
    xr_ref[...] = p[:, D_OUT:]


def _node_proj(x, w_sr):
    return pl.pallas_call(
        _proj_body,
        out_shape=(
            jax.ShapeDtypeStruct((N_NODES, D_OUT), jnp.float32),
            jax.ShapeDtypeStruct((N_NODES, D_OUT), jnp.float32),
        ),
    )(x, w_sr)


def _final_body(eat_ref, wet_ref, b_ref, *refs):
    o_ref = refs[-1]
    base = (
        jnp.dot(wet_ref[...], eat_ref[...], preferred_element_type=jnp.float32)
        + b_ref[...]
    )
    for j in range(D_OUT):
        o_ref[j:j + 1, :] = base[j:j + 1, :] + refs[j][...][None, :]


def _final(eat, wet, bcol, tmps):
    return pl.pallas_call(
        _final_body,
        grid=(pl.cdiv(N_EDGES, FIN_BLK),),
        in_specs=[
            pl.BlockSpec((D_OUT, FIN_BLK), lambda i: (0, i)),
            pl.BlockSpec((D_EDGE, D_OUT), lambda i: (0, 0)),
            pl.BlockSpec((D_OUT, 1), lambda i: (0, 0)),
        ] + [pl.BlockSpec((FIN_BLK,), lambda i: (i,)) for _ in range(D_OUT)],
        out_specs=pl.BlockSpec((D_OUT, FIN_BLK), lambda i: (0, i)),
        out_shape=jax.ShapeDtypeStruct((D_OUT, N_EDGES), jnp.float32),
    )(eat, wet, bcol, *tmps)


_SC_MESH = plsc.VectorSubcoreMesh(core_axis_name="c", subcore_axis_name="s")


@functools.partial(
    pl.kernel,
    out_type=tuple(jax.ShapeDtypeStruct((N_EDGES,), jnp.float32)
                   for _ in range(D_OUT)),
    mesh=_SC_MESH,
    compiler_params=pltpu.CompilerParams(
        use_tc_tiling_on_sc=False, needs_layout_passes=False),
    scratch_types=[
        pltpu.VMEM((2, CHUNK), jnp.int32),
        pltpu.VMEM((2, CHUNK), jnp.int32),
        pltpu.VMEM((2, SUB, D_OUT), jnp.float32),
        pltpu.VMEM((2, SUB, D_OUT), jnp.float32),
        pltpu.VMEM((D_OUT, CHUNK), jnp.float32),
        pltpu.SemaphoreType.DMA,
        pltpu.SemaphoreType.DMA,
        pltpu.SemaphoreType.DMA,
    ],
)
def _sc_edge(xs_hbm, xr_hbm, eidx_hbm, *refs):
    out_hbms = refs[:D_OUT]
    (idx_s, idx_r, bufs, bufr, buft, sems, semr, semo) = refs[D_OUT:]
    wid = lax.axis_index("s") * NC + lax.axis_index("c")
    row0 = wid * E_PER_W
    lane = lax.iota(jnp.int32, 16)
    diags = [(lane + j) % 16 for j in range(D_OUT)]

    def issue_gathers(ci, si):
        ip = ci % 2
        sp = si % 2
        sl = pl.ds(si * SUB, SUB)
        pltpu.async_copy(xs_hbm.at[idx_s.at[ip, sl]], bufs.at[sp], sems)
        pltpu.async_copy(xr_hbm.at[idx_r.at[ip, sl]], bufr.at[sp], semr)

    def wait_gathers(si):
        sp = si % 2
        pltpu.make_async_copy(xs_hbm.at[pl.ds(0, SUB)], bufs.at[sp], sems).wait()
        pltpu.make_async_copy(xr_hbm.at[pl.ds(0, SUB)], bufr.at[sp], semr).wait()

    pltpu.sync_copy(eidx_hbm.at[0, pl.ds(row0, CHUNK)], idx_s.at[0])
    pltpu.sync_copy(eidx_hbm.at[1, pl.ds(row0, CHUNK)], idx_r.at[0])
    issue_gathers(0, 0)

    for ci in range(N_CHUNKS):
        p = ci % 2
        off = row0 + ci * CHUNK
        if ci + 1 < N_CHUNKS:
            off1 = off + CHUNK
            pltpu.sync_copy(eidx_hbm.at[0, pl.ds(off1, CHUNK)],
                            idx_s.at[1 - p])
            pltpu.sync_copy(eidx_hbm.at[1, pl.ds(off1, CHUNK)],
                            idx_r.at[1 - p])
        if ci > 0:
            for j in range(D_OUT):
                pltpu.make_async_copy(buft.at[j],
                                      out_hbms[j].at[pl.ds(0, CHUNK)],
                                      semo).wait()
        for si in range(N_SUBS):
            sp = si % 2
            wait_gathers(si)
            if si + 1 < N_SUBS:
                issue_gathers(ci, si + 1)
            elif ci + 1 < N_CHUNKS:
                issue_gathers(ci + 1, 0)

            def t_body(g, c, _sp=sp, _si=si):
                rows = g * 16 + lane
                vs = [plsc.load_gather(bufs.at[_sp], [rows, d])
                      + plsc.load_gather(bufr.at[_sp], [rows, d])
                      for d in diags]
                cols = _si * SUB + rows
                for j in range(D_OUT):
                    plsc.store_scatter(buft, [diags[j], cols], vs[j])
                return c

            lax.fori_loop(0, SUB // 16, t_body, 0)
        for j in range(D_OUT):
            pltpu.async_copy(buft.at[j], out_hbms[j].at[pl.ds(off, CHUNK)],
                             semo)
    for j in range(D_OUT):
        pltpu.make_async_copy(buft.at[j], out_hbms[j].at[pl.ds(0, CHUNK)],
                              semo).wait()


def kernel(x, edge_index, edge_attr, W, b):
    w_sr = jnp.concatenate([W[:D_FEAT], W[D_FEAT:2 * D_FEAT]], axis=1)
    wet = W[2 * D_FEAT:].T
    bcol = b.reshape(D_OUT, 1)

    xs, xr = _node_proj(x, w_sr)
    tmps = _sc_edge(xs, xr, edge_index)
    out_t = _final(edge_attr.T, wet, bcol, tmps)
    return out_t.T
